# baseline (speedup 1.0000x reference)
import functools

import jax
import jax.numpy as jnp
from jax import lax
from jax.experimental import pallas as pl
from jax.experimental.pallas import tpu as pltpu
from jax.experimental.pallas import tpu_sc as plsc

_CLASSES = 100000
_DIM = 64
_BATCH = 16384
_M = 0.9

_NC = 2
_NS = 16
_L = 16
_W = _DIM + _L

_CHUNK = 12800
_PASSES = 4
_ROWS_TILE = _CHUNK // _NS
_SUB = 80
_NSUB = _ROWS_TILE // _SUB
_ITEMS = _BATCH // _NS
_GRP = 128
_NGRP = _ITEMS // _GRP

_GATHER_DNUMS = lax.GatherDimensionNumbers(
    offset_dims=(), collapsed_slice_dims=(0,), start_index_map=(0,))


def _shuf(x, idx):
    return lax.gather(x, idx[:, None], _GATHER_DNUMS, (1,),
                      mode=lax.GatherScatterMode.PROMISE_IN_BOUNDS)


def _body(feats_hbm, labels_hbm, protos_hbm, out_hbm,
          accum_sh, fbufA, fbufB, fb64A, fb64B, labels_v, idx_v,
          oldA, accA, oldB, accB,
          semz0, semz1, semz2, semfA, semfB, semstA, semstB,
          semrA1, semrA2, semrB1, semrB2, semwA, semwB):
    cid = lax.axis_index("c")
    sid = lax.axis_index("s")

    item0 = sid * _ITEMS
    pltpu.sync_copy(labels_hbm.at[pl.ds(item0, _ITEMS)], labels_v)

    zeros16 = jnp.zeros((_L,), jnp.float32)
    ones16 = jnp.ones((_L,), jnp.float32)

    def _init_ones(r, carry):
        fbufA[r, pl.ds(_DIM, _L)] = ones16
        fbufB[r, pl.ds(_DIM, _L)] = ones16
        return carry

    lax.fori_loop(0, _GRP, _init_ones, 0)

    fbufs = (fbufA, fbufB)
    fb64s = (fb64A, fb64B)
    semfs = (semfA, semfB)
    semsts = (semstA, semstB)
    semzs = (semz0, semz1, semz2)

    row0 = sid * _ROWS_TILE
    dummy = _CHUNK + sid * 8

    for p in range(_PASSES):
        base = (p * _NC + cid) * _CHUNK

        def _zero_src_row(r, carry):
            for j in range(_W // _L):
                accA[r, pl.ds(j * _L, _L)] = zeros16
            return carry

        lax.fori_loop(0, _SUB, _zero_src_row, 0)

        zdescs = [None] * (_NSUB + 1)
        for b in range(_NSUB):
            if b >= 3:
                zdescs[b - 3].wait()
            zdescs[b] = pltpu.async_copy(
                accA, accum_sh.at[pl.ds(row0 + b * _SUB, _SUB), :],
                semzs[b % 3])
        zdescs[_NSUB - 3].wait()
        zdescs[_NSUB] = pltpu.async_copy(
            accA.at[pl.ds(0, 8), :],
            accum_sh.at[pl.ds(dummy, 8), :], semzs[_NSUB % 3])

        ldesc = pltpu.async_copy(
            feats_hbm.at[pl.ds(item0, _GRP), :], fb64A, semfA)

        def _mk_idx(i, carry):
            lab = labels_v[pl.ds(i * _L, _L)]
            rel = lab - base
            valid = (rel >= 0) & (rel < _CHUNK)
            sel = jnp.where(valid, rel, dummy)
            g = i // (_GRP // _L)
            kk = (i % (_GRP // _L)) * _L
            idx_v[g, pl.ds(kk, _L)] = sel
            return carry

        lax.fori_loop(0, _ITEMS // _L, _mk_idx, 0)

        zdescs[_NSUB - 2].wait()
        zdescs[_NSUB - 1].wait()
        zdescs[_NSUB].wait()
        plsc.subcore_barrier()

        def _widen(src64, dst80):
            def _row(r, carry):
                for j in range(_DIM // _L):
                    dst80[r, pl.ds(j * _L, _L)] = src64[r, pl.ds(j * _L, _L)]
                return carry

            lax.fori_loop(0, _GRP, _row, 0)

        sdescs = [None] * _NGRP
        for g in range(_NGRP):
            cur = g % 2
            ldesc.wait()
            if g + 1 < _NGRP:
                ldesc = pltpu.async_copy(
                    feats_hbm.at[pl.ds(item0 + (g + 1) * _GRP, _GRP), :],
                    fb64s[(g + 1) % 2], semfs[(g + 1) % 2])
            if g >= 2:
                sdescs[g - 2].wait()
            _widen(fb64s[cur], fbufs[cur])
            sdescs[g] = pltpu.async_copy(
                fbufs[cur], accum_sh.at[idx_v.at[g]], semsts[g % 2],
                add=True)
        sdescs[_NGRP - 2].wait()
        sdescs[_NGRP - 1].wait()
        plsc.subcore_barrier()

        iota = lax.iota(jnp.int32, _L)
        bfly = [iota ^ k for k in (8, 4, 2, 1)]

        def _starts(b):
            g0 = base + row0 + b * _SUB
            g0s = jnp.minimum(g0, _CLASSES - _SUB)
            return g0s, g0s - base

        def _fire_reads(b, old_v, acc_v, sem1, sem2):
            g0s, l0s = _starts(b)
            return (
                pltpu.async_copy(
                    protos_hbm.at[pl.ds(g0s, _SUB), :], old_v, sem1),
                pltpu.async_copy(
                    accum_sh.at[pl.ds(l0s, _SUB), :], acc_v, sem2))

        def _compute(old_v, acc_v):
            def _fin_row(r, c2):
                o = [old_v[r, pl.ds(j * _L, _L)]
                     for j in range(_DIM // _L)]
                cnt = acc_v[r, pl.ds(_DIM, _L)]
                rowsum = (o[0] + o[1]) + (o[2] + o[3])
                for b_ix in bfly:
                    rowsum = rowsum + _shuf(rowsum, b_ix)
                anyc = cnt > 0.0
                zrow = rowsum == 0.0
                alpha = jnp.where(anyc,
                                  jnp.where(zrow, 0.0, _M), 1.0)
                beta = jnp.where(
                    anyc,
                    jnp.where(zrow, 1.0, 1.0 - _M)
                    / jnp.maximum(cnt, 1.0),
                    0.0)
                for j in range(_DIM // _L):
                    old_v[r, pl.ds(j * _L, _L)] = (
                        alpha * o[j]
                        + beta * acc_v[r, pl.ds(j * _L, _L)])
                return c2

            lax.fori_loop(0, _SUB, _fin_row, 0)

        def _fire_write(b, old_v, semw):
            g0s, _ = _starts(b)
            return pltpu.async_copy(
                old_v, out_hbm.at[pl.ds(g0s, _SUB), :], semw)

        rdA = _fire_reads(0, oldA, accA, semrA1, semrA2)
        wdA = wdB = None
        for k in range(_NSUB // 2):
            bA, bB = 2 * k, 2 * k + 1
            for d in rdA:
                d.wait()
            if wdB is not None:
                wdB.wait()
            rdB = _fire_reads(bB, oldB, accB, semrB1, semrB2)
            if wdA is not None:
                wdA.wait()
                wdA = None
            _compute(oldA, accA)
            wdA = _fire_write(bA, oldA, semwA)
            for d in rdB:
                d.wait()
            if bA + 2 < _NSUB:
                wdA.wait()
                wdA = None
                rdA = _fire_reads(bA + 2, oldA, accA, semrA1, semrA2)
            _compute(oldB, accB)
            wdB = _fire_write(bB, oldB, semwB)
        if wdA is not None:
            wdA.wait()
        wdB.wait()
        plsc.subcore_barrier()


_proto_update = functools.partial(
    pl.kernel,
    out_type=jax.ShapeDtypeStruct((_CLASSES, _DIM), jnp.float32),
    mesh=plsc.VectorSubcoreMesh(core_axis_name="c", subcore_axis_name="s"),
    compiler_params=pltpu.CompilerParams(use_tc_tiling_on_sc=False),
    scratch_types=[
        pltpu.VMEM_SHARED((_CHUNK + 8 * _NS, _W), jnp.float32),
        pltpu.VMEM((_GRP, _W), jnp.float32),
        pltpu.VMEM((_GRP, _W), jnp.float32),
        pltpu.VMEM((_GRP, _DIM), jnp.float32),
        pltpu.VMEM((_GRP, _DIM), jnp.float32),
        pltpu.VMEM((_ITEMS,), jnp.int32),
        pltpu.VMEM((_NGRP, _GRP), jnp.int32),
        pltpu.VMEM((_SUB, _DIM), jnp.float32),
        pltpu.VMEM((_SUB, _W), jnp.float32),
        pltpu.VMEM((_SUB, _DIM), jnp.float32),
        pltpu.VMEM((_SUB, _W), jnp.float32),
        pltpu.SemaphoreType.DMA,
        pltpu.SemaphoreType.DMA,
        pltpu.SemaphoreType.DMA,
        pltpu.SemaphoreType.DMA,
        pltpu.SemaphoreType.DMA,
        pltpu.SemaphoreType.DMA,
        pltpu.SemaphoreType.DMA,
        pltpu.SemaphoreType.DMA,
        pltpu.SemaphoreType.DMA,
        pltpu.SemaphoreType.DMA,
        pltpu.SemaphoreType.DMA,
        pltpu.SemaphoreType.DMA,
        pltpu.SemaphoreType.DMA,
    ],
)(_body)


def kernel(features, labels, prototypes):
    return _proto_update(features, labels.astype(jnp.int32), prototypes)


# speedup vs baseline: 1.0787x; 1.0787x over previous
import functools

import jax
import jax.numpy as jnp
from jax import lax
from jax.experimental import pallas as pl
from jax.experimental.pallas import tpu as pltpu
from jax.experimental.pallas import tpu_sc as plsc

_CLASSES = 100000
_DIM = 64
_BATCH = 16384
_M = 0.9

_NC = 2
_NS = 16
_L = 16
_W = _DIM + _L

_CHUNK = 16896
_PASSES = 3
_ROWS_TILE = _CHUNK // _NS
_SUB = 48
_NSUB = _ROWS_TILE // _SUB
_ITEMS = _BATCH // _NS
_GRP = 128
_NGRP = _ITEMS // _GRP

_GATHER_DNUMS = lax.GatherDimensionNumbers(
    offset_dims=(), collapsed_slice_dims=(0,), start_index_map=(0,))


def _shuf(x, idx):
    return lax.gather(x, idx[:, None], _GATHER_DNUMS, (1,),
                      mode=lax.GatherScatterMode.PROMISE_IN_BOUNDS)


def _body(feats_hbm, labels_hbm, protos_hbm, out_hbm,
          accum_sh, fbufA, fbufB, labels_v, idx_v,
          oldA, accA, oldB, accB,
          semz0, semz1, semz2, semfA, semfB, semstA, semstB,
          semrA1, semrA2, semrB1, semrB2, semwA, semwB):
    cid = lax.axis_index("c")
    sid = lax.axis_index("s")

    item0 = sid * _ITEMS
    pltpu.sync_copy(labels_hbm.at[pl.ds(item0, _ITEMS)], labels_v)

    zeros16 = jnp.zeros((_L,), jnp.float32)

    fbufs = (fbufA, fbufB)
    semfs = (semfA, semfB)
    semsts = (semstA, semstB)
    semzs = (semz0, semz1, semz2)

    row0 = sid * _ROWS_TILE
    dummy = _CHUNK + sid * 8

    for p in range(_PASSES):
        base = (p * _NC + cid) * _CHUNK

        def _zero_src_row(r, carry):
            for j in range(_W // _L):
                accA[r, pl.ds(j * _L, _L)] = zeros16
            return carry

        lax.fori_loop(0, _SUB, _zero_src_row, 0)

        zdescs = [None] * (_NSUB + 1)
        for b in range(_NSUB):
            if b >= 3:
                zdescs[b - 3].wait()
            zdescs[b] = pltpu.async_copy(
                accA, accum_sh.at[pl.ds(row0 + b * _SUB, _SUB), :],
                semzs[b % 3])
        zdescs[_NSUB - 3].wait()
        zdescs[_NSUB] = pltpu.async_copy(
            accA.at[pl.ds(0, 8), :],
            accum_sh.at[pl.ds(dummy, 8), :], semzs[_NSUB % 3])

        ldesc = pltpu.async_copy(
            feats_hbm.at[pl.ds(item0, _GRP), :], fbufA, semfA)

        def _mk_idx(i, carry):
            lab = labels_v[pl.ds(i * _L, _L)]
            rel = lab - base
            valid = (rel >= 0) & (rel < _CHUNK)
            sel = jnp.where(valid, rel, dummy)
            g = i // (_GRP // _L)
            kk = (i % (_GRP // _L)) * _L
            idx_v[g, pl.ds(kk, _L)] = sel
            return carry

        lax.fori_loop(0, _ITEMS // _L, _mk_idx, 0)

        zdescs[_NSUB - 2].wait()
        zdescs[_NSUB - 1].wait()
        zdescs[_NSUB].wait()
        plsc.subcore_barrier()

        sdescs = [None] * _NGRP
        for g in range(_NGRP):
            cur = fbufs[g % 2]
            ldesc.wait()
            if g + 1 < _NGRP:
                if g >= 1:
                    sdescs[g - 1].wait()
                ldesc = pltpu.async_copy(
                    feats_hbm.at[pl.ds(item0 + (g + 1) * _GRP, _GRP), :],
                    fbufs[(g + 1) % 2], semfs[(g + 1) % 2])
            sdescs[g] = pltpu.async_copy(
                cur, accum_sh.at[idx_v.at[g]], semsts[g % 2], add=True)
        sdescs[_NGRP - 2].wait()
        sdescs[_NGRP - 1].wait()
        plsc.subcore_barrier()

        iota = lax.iota(jnp.int32, _L)
        bfly = [iota ^ k for k in (8, 4, 2, 1)]

        def _starts(b):
            g0 = base + row0 + b * _SUB
            g0s = jnp.minimum(g0, _CLASSES - _SUB)
            return g0s, g0s - base

        def _fire_reads(b, old_v, acc_v, sem1, sem2):
            g0s, l0s = _starts(b)
            return (
                pltpu.async_copy(
                    protos_hbm.at[pl.ds(g0s, _SUB), :], old_v, sem1),
                pltpu.async_copy(
                    accum_sh.at[pl.ds(l0s, _SUB), :], acc_v, sem2))

        def _compute(old_v, acc_v):
            def _fin_row(r, c2):
                o = [old_v[r, pl.ds(j * _L, _L)]
                     for j in range(_DIM // _L)]
                cnt = acc_v[r, pl.ds(_DIM, _L)]
                rowsum = (o[0] + o[1]) + (o[2] + o[3])
                for b_ix in bfly:
                    rowsum = rowsum + _shuf(rowsum, b_ix)
                anyc = cnt > 0.0
                zrow = rowsum == 0.0
                alpha = jnp.where(anyc,
                                  jnp.where(zrow, 0.0, _M), 1.0)
                beta = jnp.where(
                    anyc,
                    jnp.where(zrow, 1.0, 1.0 - _M)
                    / jnp.maximum(cnt, 1.0),
                    0.0)
                for j in range(_DIM // _L):
                    old_v[r, pl.ds(j * _L, _L)] = (
                        alpha * o[j]
                        + beta * acc_v[r, pl.ds(j * _L, _L)])
                return c2

            lax.fori_loop(0, _SUB, _fin_row, 0)

        def _fire_write(b, old_v, semw):
            g0s, _ = _starts(b)
            return pltpu.async_copy(
                old_v, out_hbm.at[pl.ds(g0s, _SUB), :], semw)

        rdA = _fire_reads(0, oldA, accA, semrA1, semrA2)
        wdA = wdB = None
        for k in range(_NSUB // 2):
            bA, bB = 2 * k, 2 * k + 1
            for d in rdA:
                d.wait()
            if wdB is not None:
                wdB.wait()
            rdB = _fire_reads(bB, oldB, accB, semrB1, semrB2)
            if wdA is not None:
                wdA.wait()
                wdA = None
            _compute(oldA, accA)
            wdA = _fire_write(bA, oldA, semwA)
            for d in rdB:
                d.wait()
            if bA + 2 < _NSUB:
                wdA.wait()
                wdA = None
                rdA = _fire_reads(bA + 2, oldA, accA, semrA1, semrA2)
            _compute(oldB, accB)
            wdB = _fire_write(bB, oldB, semwB)
        if wdA is not None:
            wdA.wait()
        wdB.wait()
        plsc.subcore_barrier()


_proto_update = functools.partial(
    pl.kernel,
    out_type=jax.ShapeDtypeStruct((_CLASSES, _DIM), jnp.float32),
    mesh=plsc.VectorSubcoreMesh(core_axis_name="c", subcore_axis_name="s"),
    compiler_params=pltpu.CompilerParams(use_tc_tiling_on_sc=False),
    scratch_types=[
        pltpu.VMEM_SHARED((_CHUNK + 8 * _NS, _W), jnp.float32),
        pltpu.VMEM((_GRP, _W), jnp.float32),
        pltpu.VMEM((_GRP, _W), jnp.float32),
        pltpu.VMEM((_ITEMS,), jnp.int32),
        pltpu.VMEM((_NGRP, _GRP), jnp.int32),
        pltpu.VMEM((_SUB, _DIM), jnp.float32),
        pltpu.VMEM((_SUB, _W), jnp.float32),
        pltpu.VMEM((_SUB, _DIM), jnp.float32),
        pltpu.VMEM((_SUB, _W), jnp.float32),
        pltpu.SemaphoreType.DMA,
        pltpu.SemaphoreType.DMA,
        pltpu.SemaphoreType.DMA,
        pltpu.SemaphoreType.DMA,
        pltpu.SemaphoreType.DMA,
        pltpu.SemaphoreType.DMA,
        pltpu.SemaphoreType.DMA,
        pltpu.SemaphoreType.DMA,
        pltpu.SemaphoreType.DMA,
        pltpu.SemaphoreType.DMA,
        pltpu.SemaphoreType.DMA,
        pltpu.SemaphoreType.DMA,
        pltpu.SemaphoreType.DMA,
    ],
)(_body)


def kernel(features, labels, prototypes):
    feats80 = jnp.concatenate(
        [features, jnp.ones((features.shape[0], _L), features.dtype)], axis=1)
    return _proto_update(feats80, labels.astype(jnp.int32), prototypes)


# speedup vs baseline: 1.0881x; 1.0087x over previous
import functools

import jax
import jax.numpy as jnp
from jax import lax
from jax.experimental import pallas as pl
from jax.experimental.pallas import tpu as pltpu
from jax.experimental.pallas import tpu_sc as plsc

_CLASSES = 100000
_DIM = 64
_BATCH = 16384
_M = 0.9

_NC = 2
_NS = 16
_L = 16
_W = _DIM + _L

_CHUNK = 16896
_PASSES = 3
_ROWS_TILE = _CHUNK // _NS
_SUB = 48
_NSUB = _ROWS_TILE // _SUB
_ITEMS = _BATCH // _NS
_GRP = 128
_NGRP = _ITEMS // _GRP

_GATHER_DNUMS = lax.GatherDimensionNumbers(
    offset_dims=(), collapsed_slice_dims=(0,), start_index_map=(0,))


def _shuf(x, idx):
    return lax.gather(x, idx[:, None], _GATHER_DNUMS, (1,),
                      mode=lax.GatherScatterMode.PROMISE_IN_BOUNDS)


def _body(feats_hbm, labels_hbm, protos_hbm, out_hbm,
          accum_sh, fbufA, fbufB, labels_v, idx_v,
          oldA, accA, oldB, accB,
          semz0, semz1, semz2, semfA, semfB, semstA, semstB,
          semrA1, semrA2, semrB1, semrB2, semwA, semwB):
    cid = lax.axis_index("c")
    sid = lax.axis_index("s")

    item0 = sid * _ITEMS
    pltpu.sync_copy(labels_hbm.at[pl.ds(item0, _ITEMS)], labels_v)

    zeros16 = jnp.zeros((_L,), jnp.float32)

    fbufs = (fbufA, fbufB)
    semfs = (semfA, semfB)
    semsts = (semstA, semstB)
    semzs = (semz0, semz1, semz2)

    row0 = sid * _ROWS_TILE
    dummy = _CHUNK + sid * 8

    for p in range(_PASSES):
        base = (p * _NC + cid) * _CHUNK

        def _zero_src_row(r, carry):
            for j in range(_W // _L):
                fbufB[r, pl.ds(j * _L, _L)] = zeros16
            return carry

        lax.fori_loop(0, _GRP, _zero_src_row, 0)

        nzfull = _ROWS_TILE // _GRP
        ztail = _ROWS_TILE - nzfull * _GRP
        zdescs = [None] * (nzfull + 2)
        for b in range(nzfull):
            if b >= 3:
                zdescs[b - 3].wait()
            zdescs[b] = pltpu.async_copy(
                fbufB, accum_sh.at[pl.ds(row0 + b * _GRP, _GRP), :],
                semzs[b % 3])
        zdescs[nzfull] = pltpu.async_copy(
            fbufB.at[pl.ds(0, ztail), :],
            accum_sh.at[pl.ds(row0 + nzfull * _GRP, ztail), :],
            semzs[nzfull % 3])
        zdescs[nzfull + 1] = pltpu.async_copy(
            fbufB.at[pl.ds(0, 8), :],
            accum_sh.at[pl.ds(dummy, 8), :], semzs[(nzfull + 1) % 3])

        ldesc = pltpu.async_copy(
            feats_hbm.at[pl.ds(item0, _GRP), :], fbufA, semfA)

        def _mk_idx(i, carry):
            lab = labels_v[pl.ds(i * _L, _L)]
            rel = lab - base
            valid = (rel >= 0) & (rel < _CHUNK)
            sel = jnp.where(valid, rel, dummy)
            g = i // (_GRP // _L)
            kk = (i % (_GRP // _L)) * _L
            idx_v[g, pl.ds(kk, _L)] = sel
            return carry

        lax.fori_loop(0, _ITEMS // _L, _mk_idx, 0)

        for b in range(nzfull - 2, nzfull + 2):
            zdescs[b].wait()
        plsc.subcore_barrier()

        sdescs = [None] * _NGRP
        for g in range(_NGRP):
            cur = fbufs[g % 2]
            ldesc.wait()
            if g + 1 < _NGRP:
                if g >= 1:
                    sdescs[g - 1].wait()
                ldesc = pltpu.async_copy(
                    feats_hbm.at[pl.ds(item0 + (g + 1) * _GRP, _GRP), :],
                    fbufs[(g + 1) % 2], semfs[(g + 1) % 2])
            sdescs[g] = pltpu.async_copy(
                cur, accum_sh.at[idx_v.at[g]], semsts[g % 2], add=True)
        sdescs[_NGRP - 2].wait()
        sdescs[_NGRP - 1].wait()
        plsc.subcore_barrier()

        iota = lax.iota(jnp.int32, _L)
        bfly = [iota ^ k for k in (8, 4, 2, 1)]

        def _starts(b):
            g0 = base + row0 + b * _SUB
            g0s = jnp.minimum(g0, _CLASSES - _SUB)
            return g0s, g0s - base

        def _fire_reads(b, old_v, acc_v, sem1, sem2):
            g0s, l0s = _starts(b)
            return (
                pltpu.async_copy(
                    protos_hbm.at[pl.ds(g0s, _SUB), :], old_v, sem1),
                pltpu.async_copy(
                    accum_sh.at[pl.ds(l0s, _SUB), :], acc_v, sem2))

        def _compute(old_v, acc_v):
            def _fin_row(r, c2):
                o = [old_v[r, pl.ds(j * _L, _L)]
                     for j in range(_DIM // _L)]
                cnt = acc_v[r, pl.ds(_DIM, _L)]
                rowsum = (o[0] + o[1]) + (o[2] + o[3])
                for b_ix in bfly:
                    rowsum = rowsum + _shuf(rowsum, b_ix)
                anyc = cnt > 0.0
                zrow = rowsum == 0.0
                alpha = jnp.where(anyc,
                                  jnp.where(zrow, 0.0, _M), 1.0)
                beta = jnp.where(
                    anyc,
                    jnp.where(zrow, 1.0, 1.0 - _M)
                    / jnp.maximum(cnt, 1.0),
                    0.0)
                for j in range(_DIM // _L):
                    old_v[r, pl.ds(j * _L, _L)] = (
                        alpha * o[j]
                        + beta * acc_v[r, pl.ds(j * _L, _L)])
                return c2

            lax.fori_loop(0, _SUB, _fin_row, 0)

        def _fire_write(b, old_v, semw):
            g0s, _ = _starts(b)
            return pltpu.async_copy(
                old_v, out_hbm.at[pl.ds(g0s, _SUB), :], semw)

        rdA = _fire_reads(0, oldA, accA, semrA1, semrA2)
        wdA = wdB = None
        for k in range(_NSUB // 2):
            bA, bB = 2 * k, 2 * k + 1
            for d in rdA:
                d.wait()
            if wdB is not None:
                wdB.wait()
            rdB = _fire_reads(bB, oldB, accB, semrB1, semrB2)
            if wdA is not None:
                wdA.wait()
                wdA = None
            _compute(oldA, accA)
            wdA = _fire_write(bA, oldA, semwA)
            for d in rdB:
                d.wait()
            if bA + 2 < _NSUB:
                wdA.wait()
                wdA = None
                rdA = _fire_reads(bA + 2, oldA, accA, semrA1, semrA2)
            _compute(oldB, accB)
            wdB = _fire_write(bB, oldB, semwB)
        if wdA is not None:
            wdA.wait()
        wdB.wait()
        plsc.subcore_barrier()


_proto_update = functools.partial(
    pl.kernel,
    out_type=jax.ShapeDtypeStruct((_CLASSES, _DIM), jnp.float32),
    mesh=plsc.VectorSubcoreMesh(core_axis_name="c", subcore_axis_name="s"),
    compiler_params=pltpu.CompilerParams(use_tc_tiling_on_sc=False),
    scratch_types=[
        pltpu.VMEM_SHARED((_CHUNK + 8 * _NS, _W), jnp.float32),
        pltpu.VMEM((_GRP, _W), jnp.float32),
        pltpu.VMEM((_GRP, _W), jnp.float32),
        pltpu.VMEM((_ITEMS,), jnp.int32),
        pltpu.VMEM((_NGRP, _GRP), jnp.int32),
        pltpu.VMEM((_SUB, _DIM), jnp.float32),
        pltpu.VMEM((_SUB, _W), jnp.float32),
        pltpu.VMEM((_SUB, _DIM), jnp.float32),
        pltpu.VMEM((_SUB, _W), jnp.float32),
        pltpu.SemaphoreType.DMA,
        pltpu.SemaphoreType.DMA,
        pltpu.SemaphoreType.DMA,
        pltpu.SemaphoreType.DMA,
        pltpu.SemaphoreType.DMA,
        pltpu.SemaphoreType.DMA,
        pltpu.SemaphoreType.DMA,
        pltpu.SemaphoreType.DMA,
        pltpu.SemaphoreType.DMA,
        pltpu.SemaphoreType.DMA,
        pltpu.SemaphoreType.DMA,
        pltpu.SemaphoreType.DMA,
        pltpu.SemaphoreType.DMA,
    ],
)(_body)


def kernel(features, labels, prototypes):
    feats80 = jnp.concatenate(
        [features, jnp.ones((features.shape[0], _L), features.dtype)], axis=1)
    return _proto_update(feats80, labels.astype(jnp.int32), prototypes)


# speedup vs baseline: 1.0888x; 1.0006x over previous
import functools

import jax
import jax.numpy as jnp
from jax import lax
from jax.experimental import pallas as pl
from jax.experimental.pallas import tpu as pltpu
from jax.experimental.pallas import tpu_sc as plsc

_CLASSES = 100000
_DIM = 64
_BATCH = 16384
_M = 0.9

_NC = 2
_NS = 16
_L = 16
_W = _DIM + _L

_CHUNK = 16896
_PASSES = 3
_ROWS_TILE = _CHUNK // _NS
_SUB = 48
_NSUB = _ROWS_TILE // _SUB
_ITEMS = _BATCH // _NS
_GRP = 128
_NGRP = _ITEMS // _GRP

_GATHER_DNUMS = lax.GatherDimensionNumbers(
    offset_dims=(), collapsed_slice_dims=(0,), start_index_map=(0,))


def _shuf(x, idx):
    return lax.gather(x, idx[:, None], _GATHER_DNUMS, (1,),
                      mode=lax.GatherScatterMode.PROMISE_IN_BOUNDS)


def _body(feats_hbm, labels_hbm, protos_hbm, out_hbm,
          accum_sh, fbufA, fbufB, labels_v, idx_v,
          oldA, accA, oldB, accB,
          semz0, semz1, semz2, semfA, semfB, semstA, semstB,
          semrA1, semrA2, semrB1, semrB2, semwA, semwB):
    cid = lax.axis_index("c")
    sid = lax.axis_index("s")

    item0 = sid * _ITEMS
    pltpu.sync_copy(labels_hbm.at[pl.ds(item0, _ITEMS)], labels_v)

    zeros16 = jnp.zeros((_L,), jnp.float32)

    fbufs = (fbufA, fbufB)
    semfs = (semfA, semfB)
    semsts = (semstA, semstB)
    semzs = (semz0, semz1, semz2)

    row0 = sid * _ROWS_TILE
    dummy = _CHUNK + sid * 8

    for p in range(_PASSES):
        base = (p * _NC + cid) * _CHUNK

        def _zero_src_row(r, carry):
            for j in range(_W // _L):
                fbufB[r, pl.ds(j * _L, _L)] = zeros16
            return carry

        lax.fori_loop(0, _GRP, _zero_src_row, 0)

        nzfull = _ROWS_TILE // _GRP
        ztail = _ROWS_TILE - nzfull * _GRP
        zdescs = [None] * (nzfull + 2)
        for b in range(nzfull):
            if b >= 3:
                zdescs[b - 3].wait()
            zdescs[b] = pltpu.async_copy(
                fbufB, accum_sh.at[pl.ds(row0 + b * _GRP, _GRP), :],
                semzs[b % 3])
        zdescs[nzfull] = pltpu.async_copy(
            fbufB.at[pl.ds(0, ztail), :],
            accum_sh.at[pl.ds(row0 + nzfull * _GRP, ztail), :],
            semzs[nzfull % 3])
        zdescs[nzfull + 1] = pltpu.async_copy(
            fbufB.at[pl.ds(0, 8), :],
            accum_sh.at[pl.ds(dummy, 8), :], semzs[(nzfull + 1) % 3])

        ldesc = pltpu.async_copy(
            feats_hbm.at[pl.ds(item0, _GRP), :], fbufA, semfA)

        def _mk_idx(i, carry):
            lab = labels_v[pl.ds(i * _L, _L)]
            rel = lab - base
            valid = (rel >= 0) & (rel < _CHUNK)
            sel = jnp.where(valid, rel, dummy)
            g = i // (_GRP // _L)
            kk = (i % (_GRP // _L)) * _L
            idx_v[g, pl.ds(kk, _L)] = sel
            return carry

        lax.fori_loop(0, _ITEMS // _L, _mk_idx, 0)

        for b in range(nzfull - 2, nzfull + 2):
            zdescs[b].wait()
        plsc.subcore_barrier()

        def _pstart(b):
            g0 = base + row0 + b * _SUB
            return jnp.minimum(g0, _CLASSES - _SUB)

        pdA = pltpu.async_copy(
            protos_hbm.at[pl.ds(_pstart(0), _SUB), :], oldA, semrA1)
        pdB = pltpu.async_copy(
            protos_hbm.at[pl.ds(_pstart(1), _SUB), :], oldB, semrB1)

        sdescs = [None] * _NGRP
        for g in range(_NGRP):
            cur = fbufs[g % 2]
            ldesc.wait()
            if g + 1 < _NGRP:
                if g >= 1:
                    sdescs[g - 1].wait()
                ldesc = pltpu.async_copy(
                    feats_hbm.at[pl.ds(item0 + (g + 1) * _GRP, _GRP), :],
                    fbufs[(g + 1) % 2], semfs[(g + 1) % 2])
            sdescs[g] = pltpu.async_copy(
                cur, accum_sh.at[idx_v.at[g]], semsts[g % 2], add=True)
        sdescs[_NGRP - 2].wait()
        sdescs[_NGRP - 1].wait()
        plsc.subcore_barrier()

        iota = lax.iota(jnp.int32, _L)
        bfly = [iota ^ k for k in (8, 4, 2, 1)]

        def _starts(b):
            g0 = base + row0 + b * _SUB
            g0s = jnp.minimum(g0, _CLASSES - _SUB)
            return g0s, g0s - base

        def _fire_reads(b, old_v, acc_v, sem1, sem2):
            g0s, l0s = _starts(b)
            return (
                pltpu.async_copy(
                    protos_hbm.at[pl.ds(g0s, _SUB), :], old_v, sem1),
                pltpu.async_copy(
                    accum_sh.at[pl.ds(l0s, _SUB), :], acc_v, sem2))

        def _compute(old_v, acc_v):
            def _fin_row(r, c2):
                o = [old_v[r, pl.ds(j * _L, _L)]
                     for j in range(_DIM // _L)]
                cnt = acc_v[r, pl.ds(_DIM, _L)]
                rowsum = (o[0] + o[1]) + (o[2] + o[3])
                for b_ix in bfly:
                    rowsum = rowsum + _shuf(rowsum, b_ix)
                anyc = cnt > 0.0
                zrow = rowsum == 0.0
                alpha = jnp.where(anyc,
                                  jnp.where(zrow, 0.0, _M), 1.0)
                beta = jnp.where(
                    anyc,
                    jnp.where(zrow, 1.0, 1.0 - _M)
                    / jnp.maximum(cnt, 1.0),
                    0.0)
                for j in range(_DIM // _L):
                    old_v[r, pl.ds(j * _L, _L)] = (
                        alpha * o[j]
                        + beta * acc_v[r, pl.ds(j * _L, _L)])
                return c2

            lax.fori_loop(0, _SUB, _fin_row, 0)

        def _fire_write(b, old_v, semw):
            g0s, _ = _starts(b)
            return pltpu.async_copy(
                old_v, out_hbm.at[pl.ds(g0s, _SUB), :], semw)

        _, l0sA = _starts(0)
        rdA = (pdA, pltpu.async_copy(
            accum_sh.at[pl.ds(l0sA, _SUB), :], accA, semrA2))
        wdA = wdB = None
        for k in range(_NSUB // 2):
            bA, bB = 2 * k, 2 * k + 1
            for d in rdA:
                d.wait()
            if wdB is not None:
                wdB.wait()
            if k == 0:
                _, l0sB = _starts(1)
                rdB = (pdB, pltpu.async_copy(
                    accum_sh.at[pl.ds(l0sB, _SUB), :], accB, semrB2))
            else:
                rdB = _fire_reads(bB, oldB, accB, semrB1, semrB2)
            if wdA is not None:
                wdA.wait()
                wdA = None
            _compute(oldA, accA)
            wdA = _fire_write(bA, oldA, semwA)
            for d in rdB:
                d.wait()
            if bA + 2 < _NSUB:
                wdA.wait()
                wdA = None
                rdA = _fire_reads(bA + 2, oldA, accA, semrA1, semrA2)
            _compute(oldB, accB)
            wdB = _fire_write(bB, oldB, semwB)
        if wdA is not None:
            wdA.wait()
        wdB.wait()
        plsc.subcore_barrier()


_proto_update = functools.partial(
    pl.kernel,
    out_type=jax.ShapeDtypeStruct((_CLASSES, _DIM), jnp.float32),
    mesh=plsc.VectorSubcoreMesh(core_axis_name="c", subcore_axis_name="s"),
    compiler_params=pltpu.CompilerParams(use_tc_tiling_on_sc=False),
    scratch_types=[
        pltpu.VMEM_SHARED((_CHUNK + 8 * _NS, _W), jnp.float32),
        pltpu.VMEM((_GRP, _W), jnp.float32),
        pltpu.VMEM((_GRP, _W), jnp.float32),
        pltpu.VMEM((_ITEMS,), jnp.int32),
        pltpu.VMEM((_NGRP, _GRP), jnp.int32),
        pltpu.VMEM((_SUB, _DIM), jnp.float32),
        pltpu.VMEM((_SUB, _W), jnp.float32),
        pltpu.VMEM((_SUB, _DIM), jnp.float32),
        pltpu.VMEM((_SUB, _W), jnp.float32),
        pltpu.SemaphoreType.DMA,
        pltpu.SemaphoreType.DMA,
        pltpu.SemaphoreType.DMA,
        pltpu.SemaphoreType.DMA,
        pltpu.SemaphoreType.DMA,
        pltpu.SemaphoreType.DMA,
        pltpu.SemaphoreType.DMA,
        pltpu.SemaphoreType.DMA,
        pltpu.SemaphoreType.DMA,
        pltpu.SemaphoreType.DMA,
        pltpu.SemaphoreType.DMA,
        pltpu.SemaphoreType.DMA,
        pltpu.SemaphoreType.DMA,
    ],
)(_body)


def kernel(features, labels, prototypes):
    feats80 = jnp.concatenate(
        [features, jnp.ones((features.shape[0], _L), features.dtype)], axis=1)
    return _proto_update(feats80, labels.astype(jnp.int32), prototypes)
